# speedup vs baseline: 1.3872x; 1.3872x over previous
import functools

import jax
import jax.numpy as jnp
from jax import lax
from jax.experimental import pallas as pl
from jax.experimental.pallas import tpu as pltpu

N = 10000
E = 320000
ND = 128
H = 64
Z = 32

EBLK = 5000


def _gather_rows(table, idx):
    return jnp.take(table, idx, axis=0)


def _scatter_add(data, idx, num):
    return jax.ops.segment_sum(data, idx, num_segments=num)


def _counts(idx):
    return jax.ops.segment_sum(jnp.ones((idx.shape[0],), jnp.float32), idx,
                               num_segments=N)


def _k_node_pre(x_ref, w_ref, o_ref):
    o_ref[...] = jnp.dot(x_ref[...], w_ref[...],
                         preferred_element_type=jnp.float32)


def _node_pre(x, w_cat):
    return pl.pallas_call(
        _k_node_pre,
        out_shape=jax.ShapeDtypeStruct((N, 3 * H), jnp.float32),
    )(x, w_cat)


def _k_edge_b(gs_ref, gd_ref, ea_ref, w_ref, b_ref, o_ref):
    pre = (gs_ref[...] + gd_ref[...] + b_ref[...]
           + jnp.dot(ea_ref[...], w_ref[...],
                     preferred_element_type=jnp.float32))
    o_ref[...] = jnp.maximum(pre, 0.0)


def _edge_b(g_s, g_d, ea, w_ea, b_e):
    grid = (E // EBLK,)
    eb = lambda i: (i, 0)
    wb = lambda i: (0, 0)
    return pl.pallas_call(
        _k_edge_b,
        grid=grid,
        in_specs=[
            pl.BlockSpec((EBLK, H), eb),
            pl.BlockSpec((EBLK, H), eb),
            pl.BlockSpec((EBLK, 16), eb),
            pl.BlockSpec((16, H), wb),
            pl.BlockSpec((1, H), wb),
        ],
        out_specs=pl.BlockSpec((EBLK, H), eb),
        out_shape=jax.ShapeDtypeStruct((E, H), jnp.float32),
    )(g_s, g_d, ea, w_ea, b_e)


def _k_node_b(pxn_ref, se_ref, cnt_ref, u_ref, wn_ref, bn_ref, wg_ref, bg_ref,
              v1_ref, ic_ref, u1_ref):
    inv_cnt = 1.0 / jnp.maximum(cnt_ref[...], 1.0)
    mean_e1 = se_ref[...] * inv_cnt
    v1 = jnp.maximum(
        pxn_ref[...] + bn_ref[...]
        + jnp.dot(mean_e1, wn_ref[...], preferred_element_type=jnp.float32),
        0.0)
    v1_ref[...] = v1
    ic_ref[...] = inv_cnt
    gsum_v1 = jnp.sum(v1, axis=0, keepdims=True) / N
    gsum_e1 = jnp.sum(se_ref[...], axis=0, keepdims=True) / E
    gcat = jnp.concatenate([u_ref[...], gsum_v1, gsum_e1], axis=1)
    u1_ref[...] = jnp.maximum(
        jnp.dot(gcat, wg_ref[...], preferred_element_type=jnp.float32)
        + bg_ref[...], 0.0)


def _node_b(px_n, sum_e1, cnt, u, wn_e, bn, wg, bg):
    return pl.pallas_call(
        _k_node_b,
        out_shape=(
            jax.ShapeDtypeStruct((N, H), jnp.float32),
            jax.ShapeDtypeStruct((N, 1), jnp.float32),
            jax.ShapeDtypeStruct((1, H), jnp.float32),
        ),
    )(px_n, sum_e1, cnt, u, wn_e, bn, wg, bg)


def _k_edge_d(gvs_ref, gvd_ref, e1_ref, he_ref, wzr_ref, bzr_ref,
              wh_ref, bh_ref, o_ref):
    gvs, gvd, e1, he = gvs_ref[...], gvd_ref[...], e1_ref[...], he_ref[...]
    cat = jnp.concatenate([gvs, gvd, e1, he], axis=1)
    zr = jax.nn.sigmoid(
        jnp.dot(cat, wzr_ref[...], preferred_element_type=jnp.float32)
        + bzr_ref[...])
    z, r = zr[:, :H], zr[:, H:]
    cat2 = jnp.concatenate([gvs, gvd, e1, r * he], axis=1)
    hh = jnp.tanh(
        jnp.dot(cat2, wh_ref[...], preferred_element_type=jnp.float32)
        + bh_ref[...])
    o_ref[...] = (1.0 - z) * he + z * hh


def _edge_d(gv_s, gv_d, e1, h_e, w_zr, b_zr, w_h, b_h):
    grid = (E // EBLK,)
    eb = lambda i: (i, 0)
    wb = lambda i: (0, 0)
    return pl.pallas_call(
        _k_edge_d,
        grid=grid,
        in_specs=[
            pl.BlockSpec((EBLK, H), eb),
            pl.BlockSpec((EBLK, H), eb),
            pl.BlockSpec((EBLK, H), eb),
            pl.BlockSpec((EBLK, H), eb),
            pl.BlockSpec((4 * H, 2 * H), wb),
            pl.BlockSpec((1, 2 * H), wb),
            pl.BlockSpec((4 * H, H), wb),
            pl.BlockSpec((1, H), wb),
        ],
        out_specs=pl.BlockSpec((EBLK, H), eb),
        out_shape=jax.ShapeDtypeStruct((E, H), jnp.float32),
    )(gv_s, gv_d, e1, h_e, w_zr, b_zr, w_h, b_h)


def _k_node_d(v1_ref, sh_ref, ic_ref, hx_ref, u1_ref, hu_ref,
              wnzr_ref, bnzr_ref, wnh_ref, bnh_ref,
              wgzr_ref, bgzr_ref, wgh_ref, bgh_ref, wpm_ref,
              hx2_ref, pm_ref, hu2_ref):
    v1, hx = v1_ref[...], hx_ref[...]
    mean_he2 = sh_ref[...] * ic_ref[...]
    ncat = jnp.concatenate([v1, mean_he2, hx], axis=1)
    nzr = jax.nn.sigmoid(
        jnp.dot(ncat, wnzr_ref[...], preferred_element_type=jnp.float32)
        + bnzr_ref[...])
    nz, nr = nzr[:, :H], nzr[:, H:]
    ncat2 = jnp.concatenate([v1, mean_he2, nr * hx], axis=1)
    nhh = jnp.tanh(
        jnp.dot(ncat2, wnh_ref[...], preferred_element_type=jnp.float32)
        + bnh_ref[...])
    hx2 = (1.0 - nz) * hx + nz * nhh
    hx2_ref[...] = hx2
    pm_ref[...] = jnp.dot(hx2, wpm_ref[...],
                          preferred_element_type=jnp.float32)

    gm_hx2 = jnp.sum(hx2, axis=0, keepdims=True) / N
    gm_he2 = jnp.sum(sh_ref[...], axis=0, keepdims=True) / E
    hu = hu_ref[...]
    gcat = jnp.concatenate([u1_ref[...], gm_hx2, gm_he2, hu], axis=1)
    gzr = jax.nn.sigmoid(
        jnp.dot(gcat, wgzr_ref[...], preferred_element_type=jnp.float32)
        + bgzr_ref[...])
    gz, gr = gzr[:, :H], gzr[:, H:]
    gcat2 = jnp.concatenate([u1_ref[...], gm_hx2, gm_he2, gr * hu], axis=1)
    ghh = jnp.tanh(
        jnp.dot(gcat2, wgh_ref[...], preferred_element_type=jnp.float32)
        + bgh_ref[...])
    hu2_ref[...] = (1.0 - gz) * hu + gz * ghh


def _node_d(v1, sum_he2, inv_cnt, h_x, u1, h_u, wn_zr, bn_zr, wn_h, bn_h,
            wg_zr, bg_zr, wg_h, bg_h, w_pm):
    return pl.pallas_call(
        _k_node_d,
        out_shape=(
            jax.ShapeDtypeStruct((N, H), jnp.float32),
            jax.ShapeDtypeStruct((N, 2 * H), jnp.float32),
            jax.ShapeDtypeStruct((1, H), jnp.float32),
        ),
    )(v1, sum_he2, inv_cnt, h_x, u1, h_u, wn_zr, bn_zr, wn_h, bn_h,
      wg_zr, bg_zr, wg_h, bg_h, w_pm)


def _k_edge_f(gms_ref, gmd_ref, he2_ref, wml_ref, bml_ref, ww_ref, bw_ref,
              mu_ref, var_ref, w_ref):
    ml = (gms_ref[...] + gmd_ref[...] + bml_ref[...]
          + jnp.dot(he2_ref[...], wml_ref[...],
                    preferred_element_type=jnp.float32))
    mu = ml[:, :Z]
    mu_ref[...] = mu
    var_ref[...] = jax.nn.softplus(ml[:, Z:])
    w_ref[...] = jax.nn.sigmoid(
        jnp.dot(mu, ww_ref[...], preferred_element_type=jnp.float32)
        + bw_ref[...])


def _edge_f(gm_s, gm_d, h_e2, w_ml, b_ml, w_w, b_w):
    grid = (E // EBLK,)
    eb = lambda i: (i, 0)
    wb = lambda i: (0, 0)
    return pl.pallas_call(
        _k_edge_f,
        grid=grid,
        in_specs=[
            pl.BlockSpec((EBLK, 2 * Z), eb),
            pl.BlockSpec((EBLK, 2 * Z), eb),
            pl.BlockSpec((EBLK, H), eb),
            pl.BlockSpec((H, 2 * Z), wb),
            pl.BlockSpec((1, 2 * Z), wb),
            pl.BlockSpec((Z, 1), wb),
            pl.BlockSpec((1, 1), wb),
        ],
        out_specs=(
            pl.BlockSpec((EBLK, Z), eb),
            pl.BlockSpec((EBLK, Z), eb),
            pl.BlockSpec((EBLK, 1), eb),
        ),
        out_shape=(
            jax.ShapeDtypeStruct((E, Z), jnp.float32),
            jax.ShapeDtypeStruct((E, Z), jnp.float32),
            jax.ShapeDtypeStruct((E, 1), jnp.float32),
        ),
    )(gm_s, gm_d, h_e2, w_ml, b_ml, w_w, b_w)


def _k_node_f(hx2_ref, sz_ref, ic_ref, wml_ref, bml_ref, wg_ref, bg_ref,
              mu_ref, var_ref, g_ref):
    mean_z = sz_ref[...] * ic_ref[...]
    cat = jnp.concatenate([hx2_ref[...], mean_z], axis=1)
    ml = (jnp.dot(cat, wml_ref[...], preferred_element_type=jnp.float32)
          + bml_ref[...])
    mu = ml[:, :Z]
    mu_ref[...] = mu
    var_ref[...] = jax.nn.softplus(ml[:, Z:])
    g_ref[...] = (jnp.dot(mu, wg_ref[...], preferred_element_type=jnp.float32)
                  + bg_ref[...])


def _node_f(h_x2, sum_z, inv_cnt, w_ml, b_ml, w_g, b_g):
    return pl.pallas_call(
        _k_node_f,
        out_shape=(
            jax.ShapeDtypeStruct((N, Z), jnp.float32),
            jax.ShapeDtypeStruct((N, Z), jnp.float32),
            jax.ShapeDtypeStruct((N, 1), jnp.float32),
        ),
    )(h_x2, sum_z, inv_cnt, w_ml, b_ml, w_g, b_g)


def kernel(x, edge_index, edge_attr, u, batch, h_x, h_e, h_u, params):
    src, dst = edge_index[0], edge_index[1]

    We, be = params['enc_e']
    w_pre = jnp.concatenate([We[:ND], We[ND:2 * ND],
                             params['enc_n'][0][:ND]], axis=1)
    Wn, bn = params['enc_n']
    Wg, bg = params['enc_g']
    gz, gr, gh = params['re']['z'], params['re']['r'], params['re']['h']
    w_zr = jnp.concatenate([gz[0], gr[0]], axis=1)
    b_zr = jnp.concatenate([gz[1], gr[1]])[None]
    nz, nr, nh = params['rn']['z'], params['rn']['r'], params['rn']['h']
    wn_zr = jnp.concatenate([nz[0], nr[0]], axis=1)
    bn_zr = jnp.concatenate([nz[1], nr[1]])[None]
    gz2, gr2, gh2 = params['rg']['z'], params['rg']['r'], params['rg']['h']
    wg_zr = jnp.concatenate([gz2[0], gr2[0]], axis=1)
    bg_zr = jnp.concatenate([gz2[1], gr2[1]])[None]
    Wm, bm = params['de_mu']
    Wl, bl = params['de_lv']
    w_pm = jnp.concatenate([
        jnp.concatenate([Wm[:H], Wl[:H]], axis=1),
        jnp.concatenate([Wm[H:2 * H], Wl[H:2 * H]], axis=1)], axis=1)
    w_ml3 = jnp.concatenate([Wm[2 * H:], Wl[2 * H:]], axis=1)
    b_ml = jnp.concatenate([bm, bl])[None]
    Wdm, bdm = params['dn_mu']
    Wdl, bdl = params['dn_lv']
    w_dml = jnp.concatenate([Wdm, Wdl], axis=1)
    b_dml = jnp.concatenate([bdm, bdl])[None]

    px = _node_pre(x, w_pre)
    px_s, px_d, px_n = px[:, :H], px[:, H:2 * H], px[:, 2 * H:]
    g_s = _gather_rows(px_s, src)
    g_d = _gather_rows(px_d, dst)
    e1 = _edge_b(g_s, g_d, edge_attr, We[2 * ND:], be[None])
    sum_e1 = _scatter_add(e1, dst, N)
    cnt = _counts(dst).reshape(N, 1)
    v1, inv_cnt, u1 = _node_b(px_n, sum_e1, cnt, u, Wn[ND:], bn[None], Wg, bg[None])

    gv_s = _gather_rows(v1, src)
    gv_d = _gather_rows(v1, dst)
    h_e2 = _edge_d(gv_s, gv_d, e1, h_e, w_zr, b_zr, gh[0], gh[1][None])
    sum_he2 = _scatter_add(h_e2, dst, N)
    h_x2, pm, h_u2 = _node_d(v1, sum_he2, inv_cnt, h_x, u1, h_u,
                             wn_zr, bn_zr, nh[0], nh[1][None],
                             wg_zr, bg_zr, gh2[0], gh2[1][None], w_pm)

    pm_s, pm_d = pm[:, :H], pm[:, H:]
    gm_s = _gather_rows(pm_s, src)
    gm_d = _gather_rows(pm_d, dst)
    ze_mu, ze_var, w_ = _edge_f(gm_s, gm_d, h_e2, w_ml3, b_ml,
                                params['de_w'][0], params['de_w'][1][None])
    sum_z = _scatter_add(ze_mu, dst, N)
    zn_mu, zn_var, g_ = _node_f(h_x2, sum_z, inv_cnt, w_dml, b_dml,
                                params['dn_g'][0], params['dn_g'][1][None])

    return (h_x2, h_e2, h_u2, w_, g_, ze_mu, ze_var, zn_mu, zn_var)
